# baseline (speedup 1.0000x reference)
import functools
import math

import jax
import jax.numpy as jnp
from jax import lax
from jax.experimental import pallas as pl
from jax.experimental.pallas import tpu as pltpu
from jax.experimental.pallas import tpu_sc as plsc

_B = 64
_OBS = 128 * 1024
_FLAT = _OBS * 3

_THRESH = 2.0 * math.sin(math.radians(2.0) / 2.0)
_THRESH2 = _THRESH * _THRESH
_ALPHA = 1.0
_BETA = 1.0

_NC = 2
_NS = 16
_NW = _NC * _NS
_BPW = _B // _NW
_CHUNK = 49152
_NCH = _FLAT // _CHUNK
_TRIPS = _CHUNK // 48


def _sc_raw_score(z_hbm, r_hbm, out_hbm, buf0, buf1, rv, res, sem0, sem1):
    w = lax.axis_index("s") * _NC + lax.axis_index("c")
    b0 = w * _BPW
    pltpu.sync_copy(r_hbm, rv)

    iota = lax.iota(jnp.int32, 16)
    idx0 = iota * 3

    bufs = (buf0, buf1)
    sems = (sem0, sem1)
    sched = [(bi, ci) for bi in range(_BPW) for ci in range(_NCH)]

    def start(step):
        bi, ci = sched[step]
        k = step % 2
        return pltpu.async_copy(
            z_hbm.at[b0 + bi, ci], bufs[k], sems[k])

    copies = {0: start(0)}
    accs = [jnp.zeros((16,), jnp.float32) for _ in range(_BPW)]
    bvs = []
    for bi in range(_BPW):
        rvec = plsc.load_gather(rv, [jnp.full((16,), b0 + bi, jnp.int32)])
        bvs.append(-0.5 / (rvec * rvec))

    for step in range(len(sched)):
        bi, ci = sched[step]
        k = step % 2
        copies.pop(step).wait()
        if step + 1 < len(sched):
            copies[step + 1] = start(step + 1)
        buf = bufs[k]
        bv = bvs[bi]

        def body(i, acc):
            base = idx0 + i * 48
            xs = plsc.load_gather(buf, [base])
            ys = plsc.load_gather(buf, [base + 1])
            zs = plsc.load_gather(buf, [base + 2])
            z2 = xs * xs + ys * ys + zs * zs
            val = jnp.exp(bv * z2)
            return acc + jnp.where(z2 < _THRESH2, val, 0.0)

        accs[bi] = lax.fori_loop(0, _TRIPS, body, accs[bi])

    out_vec = jnp.zeros((16,), jnp.float32)
    for bi in range(_BPW):
        total = jnp.sum(accs[bi])
        out_vec = jnp.where(iota == bi, jnp.full((16,), total), out_vec)
    res[...] = out_vec
    pltpu.sync_copy(res, out_hbm.at[w])


@jax.jit
def _raw_score_call(z, r):
    mesh = plsc.VectorSubcoreMesh(core_axis_name="c", subcore_axis_name="s")
    f = pl.kernel(
        _sc_raw_score,
        out_type=jax.ShapeDtypeStruct((_NW, 16), jnp.float32),
        mesh=mesh,
        compiler_params=pltpu.CompilerParams(needs_layout_passes=False),
        scratch_types=[
            pltpu.VMEM((_CHUNK,), jnp.float32),
            pltpu.VMEM((_CHUNK,), jnp.float32),
            pltpu.VMEM((_B,), jnp.float32),
            pltpu.VMEM((16,), jnp.float32),
            pltpu.SemaphoreType.DMA,
            pltpu.SemaphoreType.DMA,
        ],
    )
    return f(z.reshape(_B, _NCH, _CHUNK), r)


def _tc_stats(r_ref, raw_ref, nobs_ref, mu_ref, sig2_ref, obj_ref):
    r = r_ref[...]
    a = 1.0 / (r * r)
    lam = 0.5 * a * _THRESH2
    mu_per = (1.0 - jnp.exp(-lam)) / lam
    e2 = (1.0 - jnp.exp(-2.0 * lam)) / (2.0 * lam)
    sig2_per = e2 - mu_per * mu_per
    nobs = nobs_ref[...]
    mu = nobs * mu_per
    sig2 = nobs * sig2_per
    mu_ref[...] = mu
    sig2_ref[...] = sig2
    obj_ref[...] = raw_ref[...] - _ALPHA * mu - _BETA + jnp.sqrt(sig2)


def kernel(z, R, num_obs):
    raw32 = _raw_score_call(z, R)
    raw = raw32[:, :_BPW].reshape(_B)
    shp = jax.ShapeDtypeStruct((1, _B), jnp.float32)
    mu, sig2, obj = pl.pallas_call(
        _tc_stats,
        out_shape=(shp, shp, shp),
    )(
        R.reshape(1, _B),
        raw.reshape(1, _B),
        jnp.full((1, _B), num_obs, jnp.float32),
    )
    return (raw, mu.reshape(_B), sig2.reshape(_B), obj.reshape(_B))


# speedup vs baseline: 1.0175x; 1.0175x over previous
import functools
import math

import jax
import jax.numpy as jnp
from jax import lax
from jax.experimental import pallas as pl
from jax.experimental.pallas import tpu as pltpu
from jax.experimental.pallas import tpu_sc as plsc

_B = 64
_OBS = 128 * 1024
_FLAT = _OBS * 3

_THRESH = 2.0 * math.sin(math.radians(2.0) / 2.0)
_THRESH2 = _THRESH * _THRESH
_ALPHA = 1.0
_BETA = 1.0

_NC = 2
_NS = 16
_NW = _NC * _NS
_BPW = _B // _NW
_CHUNK = 49152
_NCH = _FLAT // _CHUNK
_TRIPS = _CHUNK // 48
_UNROLL = 8


def _sc_raw_score(z_hbm, r_hbm, out_hbm, buf0, buf1, rv, res, sem0, sem1):
    w = lax.axis_index("s") * _NC + lax.axis_index("c")
    b0 = w * _BPW
    pltpu.sync_copy(r_hbm, rv)

    iota = lax.iota(jnp.int32, 16)
    idx0 = iota * 3

    bufs = (buf0, buf1)
    sems = (sem0, sem1)
    sched = [(bi, ci) for bi in range(_BPW) for ci in range(_NCH)]

    def start(step):
        bi, ci = sched[step]
        k = step % 2
        return pltpu.async_copy(
            z_hbm.at[b0 + bi, ci], bufs[k], sems[k])

    copies = {0: start(0)}
    accs = [jnp.zeros((16,), jnp.float32) for _ in range(_BPW)]
    bvs = []
    for bi in range(_BPW):
        rvec = plsc.load_gather(rv, [jnp.full((16,), b0 + bi, jnp.int32)])
        bvs.append(-0.5 / (rvec * rvec))

    for step in range(len(sched)):
        bi, ci = sched[step]
        k = step % 2
        copies.pop(step).wait()
        if step + 1 < len(sched):
            copies[step + 1] = start(step + 1)
        buf = bufs[k]
        bv = bvs[bi]

        def body(i, acc):
            vals = []
            for j in range(_UNROLL):
                base = idx0 + (i + j) * 48
                xs = plsc.load_gather(buf, [base])
                ys = plsc.load_gather(buf, [base + 1])
                zs = plsc.load_gather(buf, [base + 2])
                z2 = xs * xs + ys * ys + zs * zs
                val = jnp.exp(bv * z2)
                vals.append(jnp.where(z2 < _THRESH2, val, 0.0))
            while len(vals) > 1:
                vals = [vals[t] + vals[t + 1] for t in range(0, len(vals), 2)]
            return acc + vals[0]

        accs[bi] = plsc.parallel_loop(
            0, _TRIPS, _UNROLL, carry=accs[bi])(body)

    out_vec = jnp.zeros((16,), jnp.float32)
    for bi in range(_BPW):
        total = jnp.sum(accs[bi])
        out_vec = jnp.where(iota == bi, jnp.full((16,), total), out_vec)
    res[...] = out_vec
    pltpu.sync_copy(res, out_hbm.at[w])


@jax.jit
def _raw_score_call(z, r):
    mesh = plsc.VectorSubcoreMesh(core_axis_name="c", subcore_axis_name="s")
    f = pl.kernel(
        _sc_raw_score,
        out_type=jax.ShapeDtypeStruct((_NW, 16), jnp.float32),
        mesh=mesh,
        compiler_params=pltpu.CompilerParams(needs_layout_passes=False),
        scratch_types=[
            pltpu.VMEM((_CHUNK,), jnp.float32),
            pltpu.VMEM((_CHUNK,), jnp.float32),
            pltpu.VMEM((_B,), jnp.float32),
            pltpu.VMEM((16,), jnp.float32),
            pltpu.SemaphoreType.DMA,
            pltpu.SemaphoreType.DMA,
        ],
    )
    return f(z.reshape(_B, _NCH, _CHUNK), r)


def _tc_stats(r_ref, raw_ref, nobs_ref, mu_ref, sig2_ref, obj_ref):
    r = r_ref[...]
    a = 1.0 / (r * r)
    lam = 0.5 * a * _THRESH2
    mu_per = (1.0 - jnp.exp(-lam)) / lam
    e2 = (1.0 - jnp.exp(-2.0 * lam)) / (2.0 * lam)
    sig2_per = e2 - mu_per * mu_per
    nobs = nobs_ref[...]
    mu = nobs * mu_per
    sig2 = nobs * sig2_per
    mu_ref[...] = mu
    sig2_ref[...] = sig2
    obj_ref[...] = raw_ref[...] - _ALPHA * mu - _BETA + jnp.sqrt(sig2)


def kernel(z, R, num_obs):
    raw32 = _raw_score_call(z, R)
    raw = raw32[:, :_BPW].reshape(_B)
    shp = jax.ShapeDtypeStruct((1, _B), jnp.float32)
    mu, sig2, obj = pl.pallas_call(
        _tc_stats,
        out_shape=(shp, shp, shp),
    )(
        R.reshape(1, _B),
        raw.reshape(1, _B),
        jnp.full((1, _B), num_obs, jnp.float32),
    )
    return (raw, mu.reshape(_B), sig2.reshape(_B), obj.reshape(_B))


# speedup vs baseline: 8.3215x; 8.1786x over previous
import math

import jax
import jax.numpy as jnp
from jax import lax
from jax.experimental import pallas as pl
from jax.experimental.pallas import tpu as pltpu
from jax.experimental.pallas import tpu_sc as plsc

_B = 64
_OBS = 128 * 1024

_THRESH = 2.0 * math.sin(math.radians(2.0) / 2.0)
_THRESH2 = _THRESH * _THRESH
_ALPHA = 1.0
_BETA = 1.0

_NC = 2
_NS = 16
_NW = _NC * _NS
_BPW = _B // _NW
_SL = 2
_NST = 16 // _SL
_GRP = _SL * 64
_UNROLL = 8


def _sc_raw_score(z_hbm, r_hbm, out_hbm,
                  bx0, by0, bz0, bx1, by1, bz1, rv, res, sem0, sem1):
    w = lax.axis_index("s") * _NC + lax.axis_index("c")
    b0 = w * _BPW
    pltpu.sync_copy(r_hbm, rv)

    iota = lax.iota(jnp.int32, 16)

    bufsets = ((bx0, by0, bz0), (bx1, by1, bz1))
    sems = (sem0, sem1)
    sched = [(bi, si) for bi in range(_BPW) for si in range(_NST)]

    def start(step):
        bi, si = sched[step]
        k = step % 2
        bufs = bufsets[k]
        return [
            pltpu.async_copy(
                z_hbm.at[b0 + bi, c, pl.ds(si * _SL, _SL)], bufs[c], sems[k])
            for c in range(3)
        ]

    copies = {0: start(0)}
    accs = [jnp.zeros((16,), jnp.float32) for _ in range(_BPW)]
    bvs = []
    for bi in range(_BPW):
        rvec = plsc.load_gather(rv, [jnp.full((16,), b0 + bi, jnp.int32)])
        bvs.append(-0.5 / (rvec * rvec))

    for step in range(len(sched)):
        bi, _ = sched[step]
        k = step % 2
        for cp in copies.pop(step):
            cp.wait()
        if step + 1 < len(sched):
            copies[step + 1] = start(step + 1)
        bufx, bufy, bufz = bufsets[k]
        bv = bvs[bi]

        def body(g, acc):
            s = lax.shift_right_logical(g, 6)
            j = lax.bitwise_and(lax.shift_right_logical(g, 3), 7)
            u = lax.bitwise_and(g, 7)
            vals = []
            for lo in range(_UNROLL):
                sl = pl.ds(lo * 16, 16)
                xs = bufx[s, j, u, sl]
                ys = bufy[s, j, u, sl]
                zs = bufz[s, j, u, sl]
                z2 = xs * xs + ys * ys + zs * zs
                val = jnp.exp(bv * z2)
                vals.append(jnp.where(z2 < _THRESH2, val, 0.0))
            while len(vals) > 1:
                vals = [vals[t] + vals[t + 1] for t in range(0, len(vals), 2)]
            return acc + vals[0]

        accs[bi] = plsc.parallel_loop(0, _GRP, 1, carry=accs[bi])(body)

    out_vec = jnp.zeros((16,), jnp.float32)
    for bi in range(_BPW):
        total = jnp.sum(accs[bi])
        out_vec = jnp.where(iota == bi, jnp.full((16,), total), out_vec)
    res[...] = out_vec
    pltpu.sync_copy(res, out_hbm.at[w])


@jax.jit
def _raw_score_call(zw, r):
    mesh = plsc.VectorSubcoreMesh(core_axis_name="c", subcore_axis_name="s")
    slab = pltpu.VMEM((_SL, 8, 8, 128), jnp.float32)
    f = pl.kernel(
        _sc_raw_score,
        out_type=jax.ShapeDtypeStruct((_NW, 16), jnp.float32),
        mesh=mesh,
        compiler_params=pltpu.CompilerParams(needs_layout_passes=False),
        scratch_types=[
            slab, slab, slab, slab, slab, slab,
            pltpu.VMEM((_B,), jnp.float32),
            pltpu.VMEM((16,), jnp.float32),
            pltpu.SemaphoreType.DMA,
            pltpu.SemaphoreType.DMA,
        ],
    )
    return f(zw, r)


def _tc_stats(r_ref, raw_ref, nobs_ref, mu_ref, sig2_ref, obj_ref):
    r = r_ref[...]
    a = 1.0 / (r * r)
    lam = 0.5 * a * _THRESH2
    mu_per = (1.0 - jnp.exp(-lam)) / lam
    e2 = (1.0 - jnp.exp(-2.0 * lam)) / (2.0 * lam)
    sig2_per = e2 - mu_per * mu_per
    nobs = nobs_ref[...]
    mu = nobs * mu_per
    sig2 = nobs * sig2_per
    mu_ref[...] = mu
    sig2_ref[...] = sig2
    obj_ref[...] = raw_ref[...] - _ALPHA * mu - _BETA + jnp.sqrt(sig2)


def kernel(z, R, num_obs):
    zw = (z.transpose(0, 3, 1, 2)
           .reshape(_B, 3, 16, 8, 8, 128)
           .transpose(0, 1, 2, 4, 3, 5))
    raw32 = _raw_score_call(zw, R)
    raw = raw32[:, :_BPW].reshape(_B)
    shp = jax.ShapeDtypeStruct((1, _B), jnp.float32)
    mu, sig2, obj = pl.pallas_call(
        _tc_stats,
        out_shape=(shp, shp, shp),
    )(
        R.reshape(1, _B),
        raw.reshape(1, _B),
        jnp.full((1, _B), num_obs, jnp.float32),
    )
    return (raw, mu.reshape(_B), sig2.reshape(_B), obj.reshape(_B))


# speedup vs baseline: 8.3559x; 1.0041x over previous
import math

import jax
import jax.numpy as jnp
from jax import lax
from jax.experimental import pallas as pl
from jax.experimental.pallas import tpu as pltpu
from jax.experimental.pallas import tpu_sc as plsc

_B = 64
_OBS = 128 * 1024

_THRESH = 2.0 * math.sin(math.radians(2.0) / 2.0)
_THRESH2 = _THRESH * _THRESH
_ALPHA = 1.0
_BETA = 1.0

_NC = 2
_NS = 16
_NW = _NC * _NS
_BPW = _B // _NW
_SL = 1
_NST = 16 // _SL
_GRP = _SL * 64
_UNROLL = 8
_RING = 3


def _sc_raw_score(z_hbm, r_hbm, out_hbm,
                  bx0, by0, bz0, bx1, by1, bz1, bx2, by2, bz2,
                  rv, res, sem0, sem1, sem2):
    w = lax.axis_index("s") * _NC + lax.axis_index("c")
    b0 = w * _BPW
    pltpu.sync_copy(r_hbm, rv)

    iota = lax.iota(jnp.int32, 16)

    bufsets = ((bx0, by0, bz0), (bx1, by1, bz1), (bx2, by2, bz2))
    sems = (sem0, sem1, sem2)
    sched = [(bi, si) for bi in range(_BPW) for si in range(_NST)]

    def start(step):
        bi, si = sched[step]
        k = step % _RING
        bufs = bufsets[k]
        return [
            pltpu.async_copy(
                z_hbm.at[b0 + bi, c, si], bufs[c], sems[k])
            for c in range(3)
        ]

    copies = {s: start(s) for s in range(_RING - 1)}
    accs = [jnp.zeros((16,), jnp.float32) for _ in range(_BPW)]
    bvs = []
    for bi in range(_BPW):
        rvec = plsc.load_gather(rv, [jnp.full((16,), b0 + bi, jnp.int32)])
        bvs.append(-0.5 / (rvec * rvec))

    for step in range(len(sched)):
        bi, _ = sched[step]
        k = step % _RING
        for cp in copies.pop(step):
            cp.wait()
        if step + _RING - 1 < len(sched):
            copies[step + _RING - 1] = start(step + _RING - 1)
        bufx, bufy, bufz = bufsets[k]
        bv = bvs[bi]

        def body(g, acc):
            j = lax.shift_right_logical(g, 3)
            u = lax.bitwise_and(g, 7)
            vals = []
            for lo in range(_UNROLL):
                sl = pl.ds(lo * 16, 16)
                xs = bufx[j, u, sl]
                ys = bufy[j, u, sl]
                zs = bufz[j, u, sl]
                z2 = xs * xs + ys * ys + zs * zs
                val = jnp.exp(bv * z2)
                vals.append(jnp.where(z2 < _THRESH2, val, 0.0))
            while len(vals) > 1:
                vals = [vals[t] + vals[t + 1] for t in range(0, len(vals), 2)]
            return acc + vals[0]

        accs[bi] = plsc.parallel_loop(0, _GRP, 1, carry=accs[bi])(body)

    out_vec = jnp.zeros((16,), jnp.float32)
    for bi in range(_BPW):
        total = jnp.sum(accs[bi])
        out_vec = jnp.where(iota == bi, jnp.full((16,), total), out_vec)
    res[...] = out_vec
    pltpu.sync_copy(res, out_hbm.at[w])


@jax.jit
def _raw_score_call(zw, r):
    mesh = plsc.VectorSubcoreMesh(core_axis_name="c", subcore_axis_name="s")
    slab = pltpu.VMEM((8, 8, 128), jnp.float32)
    f = pl.kernel(
        _sc_raw_score,
        out_type=jax.ShapeDtypeStruct((_NW, 16), jnp.float32),
        mesh=mesh,
        compiler_params=pltpu.CompilerParams(needs_layout_passes=False),
        scratch_types=[
            slab, slab, slab, slab, slab, slab, slab, slab, slab,
            pltpu.VMEM((_B,), jnp.float32),
            pltpu.VMEM((16,), jnp.float32),
            pltpu.SemaphoreType.DMA,
            pltpu.SemaphoreType.DMA,
            pltpu.SemaphoreType.DMA,
        ],
    )
    return f(zw, r)


def _tc_stats(r_ref, raw_ref, nobs_ref, mu_ref, sig2_ref, obj_ref):
    r = r_ref[...]
    a = 1.0 / (r * r)
    lam = 0.5 * a * _THRESH2
    mu_per = (1.0 - jnp.exp(-lam)) / lam
    e2 = (1.0 - jnp.exp(-2.0 * lam)) / (2.0 * lam)
    sig2_per = e2 - mu_per * mu_per
    nobs = nobs_ref[...]
    mu = nobs * mu_per
    sig2 = nobs * sig2_per
    mu_ref[...] = mu
    sig2_ref[...] = sig2
    obj_ref[...] = raw_ref[...] - _ALPHA * mu - _BETA + jnp.sqrt(sig2)


def kernel(z, R, num_obs):
    zw = (z.transpose(0, 3, 1, 2)
           .reshape(_B, 3, 16, 8, 8, 128)
           .transpose(0, 1, 2, 4, 3, 5))
    raw32 = _raw_score_call(zw, R)
    raw = raw32[:, :_BPW].reshape(_B)
    shp = jax.ShapeDtypeStruct((1, _B), jnp.float32)
    mu, sig2, obj = pl.pallas_call(
        _tc_stats,
        out_shape=(shp, shp, shp),
    )(
        R.reshape(1, _B),
        raw.reshape(1, _B),
        jnp.full((1, _B), num_obs, jnp.float32),
    )
    return (raw, mu.reshape(_B), sig2.reshape(_B), obj.reshape(_B))


# speedup vs baseline: 9.4716x; 1.1335x over previous
import math

import jax
import jax.numpy as jnp
from jax import lax
from jax.experimental import pallas as pl
from jax.experimental.pallas import tpu as pltpu
from jax.experimental.pallas import tpu_sc as plsc

_B = 64
_BSC = 48
_BTC = _B - _BSC

_THRESH = 2.0 * math.sin(math.radians(2.0) / 2.0)
_THRESH2 = _THRESH * _THRESH
_ALPHA = 1.0
_BETA = 1.0

_NC = 2
_NS = 16
_NW = _NC * _NS
_UPW = _BSC * 16 // _NW
_GRP = 64
_UNROLL = 8
_RING = 3


def _sc_raw_score(z_hbm, r_hbm, out_hbm,
                  bx0, by0, bz0, bx1, by1, bz1, bx2, by2, bz2,
                  rv, res, sem0, sem1, sem2):
    w = lax.axis_index("s") * _NC + lax.axis_index("c")
    u0 = w * _UPW
    bA = lax.shift_right_logical(u0, 4)
    pltpu.sync_copy(r_hbm, rv)

    iota = lax.iota(jnp.int32, 16)

    bufsets = ((bx0, by0, bz0), (bx1, by1, bz1), (bx2, by2, bz2))
    sems = (sem0, sem1, sem2)

    def start(step):
        unit = u0 + step
        b = lax.shift_right_logical(unit, 4)
        si = lax.bitwise_and(unit, 15)
        k = step % _RING
        bufs = bufsets[k]
        return [
            pltpu.async_copy(z_hbm.at[b, c, si], bufs[c], sems[k])
            for c in range(3)
        ]

    copies = {s: start(s) for s in range(_RING - 1)}
    accA = jnp.zeros((16,), jnp.float32)
    accB = jnp.zeros((16,), jnp.float32)
    bvecA = plsc.load_gather(rv, [jnp.full((16,), bA, jnp.int32)])
    bvecB = plsc.load_gather(rv, [jnp.full((16,), bA + 1, jnp.int32)])
    bvA = -0.5 / (bvecA * bvecA)
    bvB = -0.5 / (bvecB * bvecB)

    for step in range(_UPW):
        k = step % _RING
        for cp in copies.pop(step):
            cp.wait()
        if step + _RING - 1 < _UPW:
            copies[step + _RING - 1] = start(step + _RING - 1)
        bufx, bufy, bufz = bufsets[k]
        blocal = lax.shift_right_logical(u0 + step, 4) - bA
        is_a = blocal == 0
        bv = jnp.where(is_a, bvA, bvB)

        def body(g, acc):
            j = lax.shift_right_logical(g, 3)
            u = lax.bitwise_and(g, 7)
            vals = []
            for lo in range(_UNROLL):
                sl = pl.ds(lo * 16, 16)
                xs = bufx[j, u, sl]
                ys = bufy[j, u, sl]
                zs = bufz[j, u, sl]
                z2 = xs * xs + ys * ys + zs * zs
                val = jnp.exp(bv * z2)
                vals.append(jnp.where(z2 < _THRESH2, val, 0.0))
            while len(vals) > 1:
                vals = [vals[t] + vals[t + 1] for t in range(0, len(vals), 2)]
            return acc + vals[0]

        part = plsc.parallel_loop(0, _GRP, 1,
                                  carry=jnp.zeros((16,), jnp.float32))(body)
        accA = accA + jnp.where(is_a, part, 0.0)
        accB = accB + jnp.where(is_a, 0.0, part)

    totA = jnp.sum(accA)
    totB = jnp.sum(accB)
    out_vec = jnp.where(iota == 0, jnp.full((16,), totA),
                        jnp.zeros((16,), jnp.float32))
    out_vec = jnp.where(iota == 1, jnp.full((16,), totB), out_vec)
    res[...] = out_vec
    pltpu.sync_copy(res, out_hbm.at[w])


@jax.jit
def _raw_score_call(zw6, r):
    mesh = plsc.VectorSubcoreMesh(core_axis_name="c", subcore_axis_name="s")
    slab = pltpu.VMEM((8, 8, 128), jnp.float32)
    f = pl.kernel(
        _sc_raw_score,
        out_type=jax.ShapeDtypeStruct((_NW, 16), jnp.float32),
        mesh=mesh,
        compiler_params=pltpu.CompilerParams(needs_layout_passes=False),
        scratch_types=[
            slab, slab, slab, slab, slab, slab, slab, slab, slab,
            pltpu.VMEM((_B,), jnp.float32),
            pltpu.VMEM((16,), jnp.float32),
            pltpu.SemaphoreType.DMA,
            pltpu.SemaphoreType.DMA,
            pltpu.SemaphoreType.DMA,
        ],
    )
    return f(zw6, r)


def _tc_share(bv_ref, z_ref, out_ref):
    i = pl.program_id(0)
    x = z_ref[0, 0]
    y = z_ref[0, 1]
    zc = z_ref[0, 2]
    z2 = x * x + y * y + zc * zc
    bv = bv_ref[0, i + _BSC]
    val = jnp.exp(bv * z2)
    masked = jnp.where(z2 < _THRESH2, val, 0.0)
    out_ref[0] = jnp.full((8, 128), jnp.sum(masked), jnp.float32)


def _tc_stats(r_ref, raw_ref, nobs_ref, mu_ref, sig2_ref, obj_ref):
    r = r_ref[...]
    a = 1.0 / (r * r)
    lam = 0.5 * a * _THRESH2
    mu_per = (1.0 - jnp.exp(-lam)) / lam
    e2 = (1.0 - jnp.exp(-2.0 * lam)) / (2.0 * lam)
    sig2_per = e2 - mu_per * mu_per
    nobs = nobs_ref[...]
    mu = nobs * mu_per
    sig2 = nobs * sig2_per
    mu_ref[...] = mu
    sig2_ref[...] = sig2
    obj_ref[...] = raw_ref[...] - _ALPHA * mu - _BETA + jnp.sqrt(sig2)


def kernel(z, R, num_obs):
    zw6 = (z.transpose(0, 3, 1, 2)
            .reshape(_B, 3, 16, 8, 8, 128)
            .transpose(0, 1, 2, 4, 3, 5))
    zw4 = zw6.reshape(_B, 3, 1024, 128)

    out32 = _raw_score_call(zw6, R)

    bv2d = (-0.5 / (R * R)).reshape(1, _B)
    out_tc = pl.pallas_call(
        _tc_share,
        grid=(_BTC,),
        in_specs=[
            pl.BlockSpec((1, _B), lambda i: (0, 0),
                         memory_space=pltpu.SMEM),
            pl.BlockSpec((1, 3, 1024, 128), lambda i: (i + _BSC, 0, 0, 0)),
        ],
        out_specs=pl.BlockSpec((1, 8, 128), lambda i: (i, 0, 0)),
        out_shape=jax.ShapeDtypeStruct((_BTC, 8, 128), jnp.float32),
    )(bv2d, zw4)

    e = out32[0::2, :2]
    d = out32[1::2, :2]
    raw_sc = jnp.stack([e[:, 0], e[:, 1] + d[:, 0], d[:, 1]],
                       axis=1).reshape(_BSC)
    raw = jnp.concatenate([raw_sc, out_tc[:, 0, 0]])

    shp = jax.ShapeDtypeStruct((1, _B), jnp.float32)
    mu, sig2, obj = pl.pallas_call(
        _tc_stats,
        out_shape=(shp, shp, shp),
    )(
        R.reshape(1, _B),
        raw.reshape(1, _B),
        jnp.full((1, _B), num_obs, jnp.float32),
    )
    return (raw, mu.reshape(_B), sig2.reshape(_B), obj.reshape(_B))
